# Initial kernel scaffold; baseline (speedup 1.0000x reference)
import jax
import jax.numpy as jnp
from jax.experimental import pallas as pl


def kernel(x, edge_index, edge_attr, batch, W1, b1, g1, be1, W2, b2, eps, og, ob):
    raise NotImplementedError("write your pallas kernel here")


# speedup vs baseline: 3.8863x; 3.8863x over previous
import functools

import jax
import jax.numpy as jnp
from jax import lax
from jax.experimental import pallas as pl
from jax.experimental.pallas import tpu as pltpu
from jax.experimental.pallas import tpu_sc as plsc

_LANES = 16


def _chunk_size(epw: int) -> int:
    for c in range(128, 7, -1):
        if c % 8 == 0 and epw % c == 0:
            return c
    raise ValueError(f"no valid chunk size for {epw} edges per worker")


def _sc_aggregate(h, src, dst, attr, zeros):
    NC, NS, NCH, C = src.shape
    N, D = h.shape
    KB = D // _LANES
    base_rows = (N // (8 * NS)) * 8
    rem_rows = N - NS * base_rows
    mesh = plsc.VectorSubcoreMesh(core_axis_name="core", subcore_axis_name="subcore")

    @functools.partial(
        pl.kernel,
        out_type=jax.ShapeDtypeStruct((NC, N, D), jnp.float32),
        mesh=mesh,
        scratch_types=[
            pltpu.VMEM((C,), jnp.int32),
            pltpu.VMEM((C,), jnp.int32),
            pltpu.VMEM((C,), jnp.float32),
            pltpu.VMEM((C, D), jnp.float32),
            pltpu.VMEM_SHARED((N, D), jnp.float32),
        ],
    )
    def k(h_hbm, src_hbm, dst_hbm, attr_hbm, z_hbm, out_hbm,
          src_v, dst_v, attr_v, rows_v, agg_sh):
        c = lax.axis_index("core")
        s = lax.axis_index("subcore")
        r0 = s * base_rows
        rows_sl = pl.ds(r0, base_rows)
        tail_sl = pl.ds(NS * base_rows, rem_rows)
        pltpu.sync_copy(z_hbm.at[rows_sl], agg_sh.at[rows_sl])
        if rem_rows:
            @pl.when(s == NS - 1)
            def _():
                pltpu.sync_copy(z_hbm.at[tail_sl], agg_sh.at[tail_sl])
        plsc.subcore_barrier()

        @pl.loop(0, NCH)
        def _(ci):
            pltpu.sync_copy(src_hbm.at[c, s, ci], src_v)
            pltpu.sync_copy(dst_hbm.at[c, s, ci], dst_v)
            pltpu.sync_copy(attr_hbm.at[c, s, ci], attr_v)
            pltpu.sync_copy(h_hbm.at[src_v], rows_v)

            @pl.loop(0, C // _LANES)
            def _(g):
                attr16 = attr_v[pl.ds(g * _LANES, _LANES)]
                for e in range(_LANES):
                    row = g * _LANES + e
                    w = attr16[e]
                    for kk in range(KB):
                        sl = pl.ds(kk * _LANES, _LANES)
                        rows_v[row, sl] = jnp.maximum(rows_v[row, sl] * w, 0.0)

            pltpu.sync_copy(rows_v, agg_sh.at[dst_v], add=True)

        plsc.subcore_barrier()
        pltpu.sync_copy(agg_sh.at[rows_sl], out_hbm.at[c, rows_sl])
        if rem_rows:
            @pl.when(s == NS - 1)
            def _():
                pltpu.sync_copy(agg_sh.at[tail_sl], out_hbm.at[c, tail_sl])

    return k(h, src, dst, attr, zeros)


def _tc_mlp(h, agg, scale, W1l, b1l, g1l, be1l, W2l, b2l, ogl, obl, act):
    N, D = h.shape

    def body(scale_ref, h_ref, a_ref, W1_ref, b1_ref, g1_ref, be1_ref,
             W2_ref, b2_ref, og_ref, ob_ref, out_ref):
        z = h_ref[...] * scale_ref[...] + a_ref[0] + a_ref[1]
        y = jnp.dot(z, W1_ref[...], preferred_element_type=jnp.float32) + b1_ref[...]
        m = jnp.mean(y, axis=0, keepdims=True)
        v = jnp.mean((y - m) ** 2, axis=0, keepdims=True)
        y = jnp.maximum((y - m) * lax.rsqrt(v + 1e-5) * g1_ref[...] + be1_ref[...], 0.0)
        u = jnp.dot(y, W2_ref[...], preferred_element_type=jnp.float32) + b2_ref[...]
        m2 = jnp.mean(u, axis=0, keepdims=True)
        v2 = jnp.mean((u - m2) ** 2, axis=0, keepdims=True)
        o = (u - m2) * lax.rsqrt(v2 + 1e-5) * og_ref[...] + ob_ref[...]
        if act:
            o = jnp.maximum(o, 0.0)
        out_ref[...] = o

    return pl.pallas_call(
        body,
        out_shape=jax.ShapeDtypeStruct((N, D), jnp.float32),
    )(scale, h, agg, W1l, b1l, g1l, be1l, W2l, b2l, ogl, obl)


def kernel(x, edge_index, edge_attr, batch, W1, b1, g1, be1, W2, b2, eps, og, ob):
    N, D = x.shape
    E = edge_attr.shape[0]
    L = W1.shape[0]
    NC, NS = 2, 16
    epw = E // (NC * NS)
    C = _chunk_size(epw)
    NCH = epw // C

    src = edge_index[0].reshape(NC, NS, NCH, C)
    dst = edge_index[1].reshape(NC, NS, NCH, C)
    attr = edge_attr.reshape(NC, NS, NCH, C)
    zeros = jnp.zeros((N, D), jnp.float32)

    h = x
    hs = [x]
    for l in range(L):
        agg = _sc_aggregate(h, src, dst, attr, zeros)
        scale = (1.0 + eps[l]).reshape(1, 1)
        h = _tc_mlp(h, agg, scale,
                    W1[l], b1[l].reshape(1, -1), g1[l].reshape(1, -1),
                    be1[l].reshape(1, -1), W2[l], b2[l].reshape(1, -1),
                    og[l].reshape(1, -1), ob[l].reshape(1, -1),
                    act=(l < L - 1))
        hs.append(h)
    return (hs[-1], *hs)


# speedup vs baseline: 8.4941x; 2.1856x over previous
import functools

import jax
import jax.numpy as jnp
from jax import lax
from jax.experimental import pallas as pl
from jax.experimental.pallas import tpu as pltpu
from jax.experimental.pallas import tpu_sc as plsc

_LANES = 16


def _chunk_geometry(epw: int):
    for c in (50, 80, 100, 64, 40, 25, 20, 16):
        if epw % c:
            continue
        nch = epw // c
        for nb in (20, 16, 10, 8, 4, 2):
            if nch % nb == 0:
                return c, nb, nch // nb
    raise ValueError(f"no valid chunk geometry for {epw} edges per worker")


def _sc_aggregate(h, src, dst, attr, zeros):
    NC, NS, NBLK, NB, C = src.shape
    N, D = h.shape
    goffs = [g * _LANES for g in range(C // _LANES)]
    if C % _LANES:
        goffs.append(C - _LANES)
    base_rows = (N // (8 * NS)) * 8
    rem_rows = N - NS * base_rows
    mesh = plsc.VectorSubcoreMesh(core_axis_name="core", subcore_axis_name="subcore")

    @functools.partial(
        pl.kernel,
        out_type=jax.ShapeDtypeStruct((NC, N, D), jnp.float32),
        mesh=mesh,
        scratch_types=[
            pltpu.VMEM((NB, C), jnp.int32),
            pltpu.VMEM((NB, C), jnp.int32),
            pltpu.VMEM((NB, C), jnp.float32),
            pltpu.VMEM((C, D), jnp.float32),
            pltpu.VMEM((C, D), jnp.float32),
            pltpu.VMEM((C, D), jnp.float32),
            pltpu.VMEM((C, D), jnp.float32),
            pltpu.VMEM_SHARED((N, D), jnp.float32),
            pltpu.SemaphoreType.DMA,
            pltpu.SemaphoreType.DMA,
            pltpu.SemaphoreType.DMA,
            pltpu.SemaphoreType.DMA,
        ],
    )
    def k(h_hbm, src_hbm, dst_hbm, attr_hbm, z_hbm, out_hbm,
          src_v, dst_v, attr_v, gA, gB, sA, sB, agg_sh,
          gsemA, gsemB, ssemA, ssemB):
        c = lax.axis_index("core")
        s = lax.axis_index("subcore")
        r0 = s * base_rows
        rows_sl = pl.ds(r0, base_rows)
        tail_sl = pl.ds(NS * base_rows, rem_rows)
        pltpu.sync_copy(z_hbm.at[rows_sl], agg_sh.at[rows_sl])
        if rem_rows:
            @pl.when(s == NS - 1)
            def _():
                pltpu.sync_copy(z_hbm.at[tail_sl], agg_sh.at[tail_sl])
        plsc.subcore_barrier()

        def scale(gbuf, sbuf, p):
            for off in goffs:
                attr16 = attr_v[p, pl.ds(off, _LANES)]
                for e in range(_LANES):
                    w = attr16[e]
                    for kk in range(0, D, _LANES):
                        sl = pl.ds(kk, _LANES)
                        sbuf[off + e, sl] = jnp.maximum(gbuf[off + e, sl] * w, 0.0)

        @pl.loop(0, NBLK)
        def _(b):
            @pl.when(b > 0)
            def _():
                pltpu.make_async_copy(sA, agg_sh.at[dst_v.at[0]], ssemA).wait()
                pltpu.make_async_copy(sB, agg_sh.at[dst_v.at[0]], ssemB).wait()
            pltpu.sync_copy(src_hbm.at[c, s, b], src_v)
            pltpu.sync_copy(dst_hbm.at[c, s, b], dst_v)
            pltpu.sync_copy(attr_hbm.at[c, s, b], attr_v)
            pltpu.async_copy(h_hbm.at[src_v.at[0]], gA, gsemA)

            @pl.loop(0, NB, step=2)
            def _(p):
                pltpu.async_copy(h_hbm.at[src_v.at[p + 1]], gB, gsemB)
                pltpu.make_async_copy(h_hbm.at[src_v.at[p]], gA, gsemA).wait()
                @pl.when(p > 0)
                def _():
                    pltpu.make_async_copy(sA, agg_sh.at[dst_v.at[p]], ssemA).wait()
                scale(gA, sA, p)
                pltpu.async_copy(sA, agg_sh.at[dst_v.at[p]], ssemA, add=True)
                @pl.when(p + 2 < NB)
                def _():
                    pltpu.async_copy(h_hbm.at[src_v.at[p + 2]], gA, gsemA)
                pltpu.make_async_copy(h_hbm.at[src_v.at[p + 1]], gB, gsemB).wait()
                @pl.when(p > 0)
                def _():
                    pltpu.make_async_copy(sB, agg_sh.at[dst_v.at[p]], ssemB).wait()
                scale(gB, sB, p + 1)
                pltpu.async_copy(sB, agg_sh.at[dst_v.at[p + 1]], ssemB, add=True)

        pltpu.make_async_copy(sA, agg_sh.at[dst_v.at[0]], ssemA).wait()
        pltpu.make_async_copy(sB, agg_sh.at[dst_v.at[0]], ssemB).wait()
        plsc.subcore_barrier()
        pltpu.sync_copy(agg_sh.at[rows_sl], out_hbm.at[c, rows_sl])
        if rem_rows:
            @pl.when(s == NS - 1)
            def _():
                pltpu.sync_copy(agg_sh.at[tail_sl], out_hbm.at[c, tail_sl])

    return k(h, src, dst, attr, zeros)


def _tc_mlp(h, agg, scale, W1l, b1l, g1l, be1l, W2l, b2l, ogl, obl, act):
    N, D = h.shape

    def body(scale_ref, h_ref, a_ref, W1_ref, b1_ref, g1_ref, be1_ref,
             W2_ref, b2_ref, og_ref, ob_ref, out_ref):
        z = h_ref[...] * scale_ref[...] + a_ref[0] + a_ref[1]
        y = jnp.dot(z, W1_ref[...], preferred_element_type=jnp.float32) + b1_ref[...]
        m = jnp.mean(y, axis=0, keepdims=True)
        v = jnp.mean((y - m) ** 2, axis=0, keepdims=True)
        y = jnp.maximum((y - m) * lax.rsqrt(v + 1e-5) * g1_ref[...] + be1_ref[...], 0.0)
        u = jnp.dot(y, W2_ref[...], preferred_element_type=jnp.float32) + b2_ref[...]
        m2 = jnp.mean(u, axis=0, keepdims=True)
        v2 = jnp.mean((u - m2) ** 2, axis=0, keepdims=True)
        o = (u - m2) * lax.rsqrt(v2 + 1e-5) * og_ref[...] + ob_ref[...]
        if act:
            o = jnp.maximum(o, 0.0)
        out_ref[...] = o

    return pl.pallas_call(
        body,
        out_shape=jax.ShapeDtypeStruct((N, D), jnp.float32),
    )(scale, h, agg, W1l, b1l, g1l, be1l, W2l, b2l, ogl, obl)


def kernel(x, edge_index, edge_attr, batch, W1, b1, g1, be1, W2, b2, eps, og, ob):
    N, D = x.shape
    E = edge_attr.shape[0]
    L = W1.shape[0]
    NC, NS = 2, 16
    epw = E // (NC * NS)
    C, NB, NBLK = _chunk_geometry(epw)

    src = edge_index[0].reshape(NC, NS, NBLK, NB, C)
    dst = edge_index[1].reshape(NC, NS, NBLK, NB, C)
    attr = edge_attr.reshape(NC, NS, NBLK, NB, C)
    zeros = jnp.zeros((N, D), jnp.float32)

    h = x
    hs = [x]
    for l in range(L):
        agg = _sc_aggregate(h, src, dst, attr, zeros)
        scale = (1.0 + eps[l]).reshape(1, 1)
        h = _tc_mlp(h, agg, scale,
                    W1[l], b1[l].reshape(1, -1), g1[l].reshape(1, -1),
                    be1[l].reshape(1, -1), W2[l], b2[l].reshape(1, -1),
                    og[l].reshape(1, -1), ob[l].reshape(1, -1),
                    act=(l < L - 1))
        hs.append(h)
    return (hs[-1], *hs)


# speedup vs baseline: 9.2244x; 1.0860x over previous
import dataclasses
import functools

import jax
import jax.numpy as jnp
from jax import lax
from jax.experimental import pallas as pl
from jax.experimental.pallas import tpu as pltpu
from jax.experimental.pallas import tpu_sc as plsc

_LANES = 16


def _chunk_geometry(epw: int):
    for c in (50, 80, 100, 64, 40, 25, 20, 16):
        if epw % c:
            continue
        nch = epw // c
        for nb in (20, 16, 10, 8, 4, 2):
            if nch % nb == 0:
                return c, nb, nch // nb
    raise ValueError(f"no valid chunk geometry for {epw} edges per worker")


def _sc_aggregate(h, eidx, zeros):
    NC, NS, NBLK, NB3, C = eidx.shape
    NB = NB3 // 3
    N, D = h.shape
    goffs = [g * _LANES for g in range(C // _LANES)]
    if C % _LANES:
        goffs.append(C - _LANES)
    base_rows = (N // (8 * NS)) * 8
    rem_rows = N - NS * base_rows
    mesh = plsc.VectorSubcoreMesh(core_axis_name="core", subcore_axis_name="subcore")
    cp = pltpu.CompilerParams()
    if "needs_layout_passes" in pltpu.CompilerParams.__dataclass_fields__:
        cp = dataclasses.replace(cp, needs_layout_passes=False)

    @functools.partial(
        pl.kernel,
        out_type=jax.ShapeDtypeStruct((NC, N, D), jnp.float32),
        mesh=mesh,
        compiler_params=cp,
        scratch_types=[
            pltpu.VMEM((3 * NB, C), jnp.int32),
            pltpu.VMEM((3 * NB, C), jnp.int32),
            pltpu.VMEM((C, D), jnp.float32),
            pltpu.VMEM((C, D), jnp.float32),
            pltpu.VMEM((C, D), jnp.float32),
            pltpu.VMEM((C, D), jnp.float32),
            pltpu.VMEM_SHARED((N, D), jnp.float32),
            pltpu.SemaphoreType.DMA,
            pltpu.SemaphoreType.DMA,
            pltpu.SemaphoreType.DMA,
            pltpu.SemaphoreType.DMA,
            pltpu.SemaphoreType.DMA,
        ],
    )
    def k(h_hbm, eidx_hbm, z_hbm, out_hbm,
          I0, I1, gA, gB, sA, sB, agg_sh,
          gsemA, gsemB, ssemA, ssemB, isem):
        c = lax.axis_index("core")
        s = lax.axis_index("subcore")
        r0 = s * base_rows
        rows_sl = pl.ds(r0, base_rows)
        tail_sl = pl.ds(NS * base_rows, rem_rows)
        pltpu.sync_copy(z_hbm.at[rows_sl], agg_sh.at[rows_sl])
        if rem_rows:
            @pl.when(s == NS - 1)
            def _():
                pltpu.sync_copy(z_hbm.at[tail_sl], agg_sh.at[tail_sl])
        pltpu.sync_copy(eidx_hbm.at[c, s, 0], I0)
        plsc.subcore_barrier()

        def scale(I, gbuf, sbuf, p):
            for off in goffs:
                attr16 = plsc.bitcast(I[2 * NB + p, pl.ds(off, _LANES)], jnp.float32)
                for e in range(_LANES):
                    w = attr16[e]
                    for kk in range(0, D, _LANES):
                        sl = pl.ds(kk, _LANES)
                        sbuf[off + e, sl] = jnp.maximum(gbuf[off + e, sl] * w, 0.0)

        def run_block(I, Inext, b, first, pf_b, pf_cond):
            pltpu.async_copy(h_hbm.at[I.at[0]], gA, gsemA)

            @pl.loop(0, NB, step=2)
            def _(p):
                pltpu.async_copy(h_hbm.at[I.at[p + 1]], gB, gsemB)
                pltpu.make_async_copy(h_hbm.at[I.at[p]], gA, gsemA).wait()
                @pl.when(jnp.logical_and(p == 2, pf_cond))
                def _():
                    pltpu.async_copy(eidx_hbm.at[c, s, pf_b], Inext, isem)
                @pl.when(jnp.logical_or(p > 0, jnp.logical_not(first)))
                def _():
                    pltpu.make_async_copy(sA, agg_sh.at[I.at[NB + p]], ssemA).wait()
                scale(I, gA, sA, p)
                pltpu.async_copy(sA, agg_sh.at[I.at[NB + p]], ssemA, add=True)
                @pl.when(p + 2 < NB)
                def _():
                    pltpu.async_copy(h_hbm.at[I.at[p + 2]], gA, gsemA)
                pltpu.make_async_copy(h_hbm.at[I.at[p + 1]], gB, gsemB).wait()
                @pl.when(jnp.logical_or(p > 0, jnp.logical_not(first)))
                def _():
                    pltpu.make_async_copy(sB, agg_sh.at[I.at[NB + p]], ssemB).wait()
                scale(I, gB, sB, p + 1)
                pltpu.async_copy(sB, agg_sh.at[I.at[NB + p + 1]], ssemB, add=True)

        @pl.loop(0, NBLK, step=2)
        def _(b):
            t = jnp.bool_(True)
            run_block(I0, I1, b, first=(b == 0), pf_b=b + 1, pf_cond=t)
            pltpu.make_async_copy(eidx_hbm.at[c, s, b + 1], I1, isem).wait()
            run_block(I1, I0, b + 1, first=jnp.bool_(False),
                      pf_b=b + 2, pf_cond=(b + 2 < NBLK))
            @pl.when(b + 2 < NBLK)
            def _():
                pltpu.make_async_copy(eidx_hbm.at[c, s, 0], I0, isem).wait()

        pltpu.make_async_copy(sA, agg_sh.at[I1.at[NB]], ssemA).wait()
        pltpu.make_async_copy(sB, agg_sh.at[I1.at[NB]], ssemB).wait()
        plsc.subcore_barrier()
        pltpu.sync_copy(agg_sh.at[rows_sl], out_hbm.at[c, rows_sl])
        if rem_rows:
            @pl.when(s == NS - 1)
            def _():
                pltpu.sync_copy(agg_sh.at[tail_sl], out_hbm.at[c, tail_sl])

    return k(h, eidx, zeros)


def _tc_mlp(h, agg, scale, W1l, b1l, g1l, be1l, W2l, b2l, ogl, obl, act):
    N, D = h.shape

    def body(scale_ref, h_ref, a_ref, W1_ref, b1_ref, g1_ref, be1_ref,
             W2_ref, b2_ref, og_ref, ob_ref, out_ref):
        z = h_ref[...] * scale_ref[...] + a_ref[0] + a_ref[1]
        y = jnp.dot(z, W1_ref[...], preferred_element_type=jnp.float32) + b1_ref[...]
        m = jnp.mean(y, axis=0, keepdims=True)
        v = jnp.mean((y - m) ** 2, axis=0, keepdims=True)
        y = jnp.maximum((y - m) * lax.rsqrt(v + 1e-5) * g1_ref[...] + be1_ref[...], 0.0)
        u = jnp.dot(y, W2_ref[...], preferred_element_type=jnp.float32) + b2_ref[...]
        m2 = jnp.mean(u, axis=0, keepdims=True)
        v2 = jnp.mean((u - m2) ** 2, axis=0, keepdims=True)
        o = (u - m2) * lax.rsqrt(v2 + 1e-5) * og_ref[...] + ob_ref[...]
        if act:
            o = jnp.maximum(o, 0.0)
        out_ref[...] = o

    return pl.pallas_call(
        body,
        out_shape=jax.ShapeDtypeStruct((N, D), jnp.float32),
    )(scale, h, agg, W1l, b1l, g1l, be1l, W2l, b2l, ogl, obl)


def kernel(x, edge_index, edge_attr, batch, W1, b1, g1, be1, W2, b2, eps, og, ob):
    N, D = x.shape
    E = edge_attr.shape[0]
    L = W1.shape[0]
    NC, NS = 2, 16
    epw = E // (NC * NS)
    C, NB, NBLK = _chunk_geometry(epw)

    packed = jnp.stack([edge_index[0], edge_index[1],
                        jax.lax.bitcast_convert_type(edge_attr, jnp.int32)])
    eidx = jnp.transpose(packed.reshape(3, NC, NS, NBLK, NB * C),
                         (1, 2, 3, 0, 4)).reshape(NC, NS, NBLK, 3 * NB, C)
    zeros = jnp.zeros((N, D), jnp.float32)

    h = x
    hs = [x]
    for l in range(L):
        agg = _sc_aggregate(h, eidx, zeros)
        scale = (1.0 + eps[l]).reshape(1, 1)
        h = _tc_mlp(h, agg, scale,
                    W1[l], b1[l].reshape(1, -1), g1[l].reshape(1, -1),
                    be1[l].reshape(1, -1), W2[l], b2[l].reshape(1, -1),
                    og[l].reshape(1, -1), ob[l].reshape(1, -1),
                    act=(l < L - 1))
        hs.append(h)
    return (hs[-1], *hs)
